# baseline (speedup 1.0000x reference)
import jax
import jax.numpy as jnp
from jax.experimental import pallas as pl


def _l2_min_body(ze_ref, emb_ref, out_ref):
    e = emb_ref[...]
    en = (e * -2.0).astype(jnp.bfloat16)
    ee = jnp.sum(e * e, axis=1, keepdims=True)
    B = ze_ref.shape[0]
    for b in range(B):
        z = ze_ref[b]
        dot = jax.lax.dot_general(
            en, z.astype(jnp.bfloat16), (((1,), (0,)), ((), ())),
            preferred_element_type=jnp.float32,
        )
        zz = jnp.sum(z * z, axis=0)
        out_ref[b, :] = jnp.min(dot + ee, axis=0) + zz


def kernel(ze, emb):
    B, Q, N = ze.shape
    K, _ = emb.shape
    return pl.pallas_call(
        _l2_min_body,
        out_shape=jax.ShapeDtypeStruct((B, N), jnp.float32),
    )(ze, emb)
